# baseline (speedup 1.0000x reference)
import functools

import jax
import jax.numpy as jnp
from jax import lax
from jax.experimental import pallas as pl
from jax.experimental.pallas import tpu as pltpu
from jax.experimental.pallas import tpu_sc as plsc

_MESH = plsc.VectorSubcoreMesh(core_axis_name="c", subcore_axis_name="s")
_CP = pltpu.CompilerParams(needs_layout_passes=False)


def _phase1(B, D, NC, NS, CAP, V):
    NW = NC * NS
    NCOLS = V // 128
    base_cols = NCOLS // NW
    extra = NCOLS - base_cols * NW
    NBLK = CAP // 128
    blk_base = NBLK // NS
    blk_extra = NBLK - blk_base * NS

    @functools.partial(
        pl.kernel,
        mesh=_MESH,
        out_type=(
            jax.ShapeDtypeStruct((D + 1, CAP), jnp.float32),
            jax.ShapeDtypeStruct((D + 1, CAP), jnp.float32),
        ),
        scratch_types=[
            pltpu.VMEM((2048,), jnp.int32),
            pltpu.VMEM((B,), jnp.int32),
            pltpu.VMEM((B,), jnp.int32),
            pltpu.VMEM((16,), jnp.int32),
            pltpu.VMEM((16,), jnp.int32),
            pltpu.VMEM((D, 128), jnp.float32),
            pltpu.VMEM((D, 128), jnp.float32),
            pltpu.VMEM((D + 1, 128), jnp.float32),
            pltpu.SMEM((1,), jnp.int32),
            pltpu.SemaphoreType.DMA,
            pltpu.SemaphoreType.DMA,
        ],
        compiler_params=_CP,
    )
    def phase1(
        idx_hbm, t_hbm, rec0_hbm, rec1_hbm,
        idx_v, lv, lj, t16l, t16j, blk0, blk1, batch,
        cnt_s, semb0, semb1,
    ):
        co = lax.axis_index("c")
        sid = lax.axis_index("s")
        wid = co * NS + sid
        c0 = wid * base_cols + jnp.minimum(wid, extra)
        ncol = base_cols + jnp.where(wid < extra, 1, 0)
        lanes = lax.iota(jnp.int32, 16)
        sentv = plsc.bitcast(jnp.full((16,), B, jnp.int32), jnp.float32)

        cnt_s[0] = 0
        t16l[pl.ds(0, 16)] = jnp.zeros((16,), jnp.int32)
        t16j[pl.ds(0, 16)] = jnp.zeros((16,), jnp.int32)

        def scan_win(wi, n):
            pltpu.sync_copy(idx_hbm.at[pl.ds(wi * 2048, 2048)], idx_v)

            def scan_body(k2, n):
                v = idx_v[pl.ds(k2 * 16, 16)]
                c = v // 128
                m = (c >= c0) & (c < c0 + ncol)
                cnt = jnp.sum(jnp.where(m, 1, 0))
                plsc.store_compressed(lv.at[pl.ds(n, 16)], v, mask=m)
                plsc.store_compressed(
                    lj.at[pl.ds(n, 16)], lanes + wi * 2048 + k2 * 16, mask=m
                )
                return n + cnt

            return lax.fori_loop(0, 2048 // 16, scan_body, n)

        nmatch = lax.fori_loop(0, B // 2048, scan_win, jnp.int32(0))

        def fill_sent(k, _):
            batch[D, pl.ds(k * 16, 16)] = sentv
            return 0

        lax.fori_loop(0, 128 // 16, fill_sent, 0)

        s0 = sid * blk_base + jnp.minimum(sid, blk_extra)
        nsb = blk_base + jnp.where(sid < blk_extra, 1, 0)

        def sent_blk(b, _):
            off = pl.multiple_of((s0 + b) * 128, 128)

            @pl.when(co == 0)
            def _():
                pltpu.sync_copy(
                    batch.at[:, pl.ds(0, 128)], rec0_hbm.at[:, pl.ds(off, 128)]
                )

            @pl.when(co == 1)
            def _():
                pltpu.sync_copy(
                    batch.at[:, pl.ds(0, 128)], rec1_hbm.at[:, pl.ds(off, 128)]
                )

            return 0

        lax.fori_loop(0, nsb, sent_blk, 0)
        plsc.subcore_barrier()

        nround = ((nmatch + 127) // 128) * 128
        my_base = plsc.fetch_and_add(cnt_s.at[0], nround, subcore_id=0)

        def flush(nflushed):
            off = pl.multiple_of(my_base + nflushed, 128)

            @pl.when(co == 0)
            def _():
                pltpu.sync_copy(
                    batch.at[:, pl.ds(0, 128)], rec0_hbm.at[:, pl.ds(off, 128)]
                )

            @pl.when(co == 1)
            def _():
                pltpu.sync_copy(
                    batch.at[:, pl.ds(0, 128)], rec1_hbm.at[:, pl.ds(off, 128)]
                )

        def block_body(t, carry):
            off_t = pl.multiple_of((c0 + t) * 128, 128)
            pltpu.sync_copy(t_hbm.at[:, pl.ds(off_t, 128)], blk0)

            def collect(q, carry2):
                s, nflushed = carry2
                v = lv[pl.ds(q * 16, 16)]
                j = lj[pl.ds(q * 16, 16)]
                active = (lanes + q * 16) < nmatch
                m = active & (v // 128 == c0 + t)
                cnt = jnp.sum(jnp.where(m, 1, 0))
                plsc.store_compressed(t16l.at[pl.ds(0, 16)], v % 128, mask=m)
                plsc.store_compressed(t16j.at[pl.ds(0, 16)], j, mask=m)

                def write_part(shift):
                    lvec = t16l[pl.ds(0, 16)]
                    jvec = t16j[pl.ds(0, 16)]
                    dst = s + lanes - shift
                    wm = (lanes < cnt) & (dst >= 0) & (dst < 128)
                    dstc = jnp.clip(dst, 0, 127)

                    def frow(f, _):
                        fv = jnp.full((16,), 0, jnp.int32) + f
                        col = plsc.load_gather(blk0, [fv, lvec], mask=wm)
                        plsc.store_scatter(batch, [fv, dstc], col, mask=wm)
                        return 0

                    lax.fori_loop(0, D, frow, 0)
                    plsc.store_scatter(
                        batch, [jnp.full((16,), D, jnp.int32), dstc],
                        plsc.bitcast(jvec, jnp.float32), mask=wm,
                    )

                @pl.when(cnt > 0)
                def _():
                    write_part(0)

                def do_flush(carry3):
                    s, nflushed = carry3
                    flush(nflushed)

                    def resent(k, _):
                        batch[D, pl.ds(k * 16, 16)] = sentv
                        return 0

                    lax.fori_loop(0, 128 // 16, resent, 0)

                    @pl.when(s + cnt > 128)
                    def _():
                        write_part(128)

                    return s + cnt - 128, nflushed + 128

                return lax.cond(
                    s + cnt >= 128, do_flush,
                    lambda c: (c[0] + cnt, c[1]), (s, nflushed)
                )

            nq = (nmatch + 15) // 16
            return lax.fori_loop(0, nq, collect, carry)

        sfin, nfl = lax.fori_loop(
            0, ncol, block_body, (jnp.int32(0), jnp.int32(0))
        )

        @pl.when(sfin > 0)
        def _():
            flush(nfl)

    return phase1


def _phase2(B, D, NC, NS, CAP):
    SLAB = B // NS

    @functools.partial(
        pl.kernel,
        mesh=_MESH,
        out_type=(
            jax.ShapeDtypeStruct((D, B), jnp.float32),
            jax.ShapeDtypeStruct((D, B), jnp.float32),
        ),
        scratch_types=[
            pltpu.VMEM((D, SLAB), jnp.float32),
            pltpu.VMEM((D + 1, 128), jnp.float32),
            pltpu.VMEM((144,), jnp.int32),
            pltpu.VMEM((144,), jnp.int32),
        ],
        compiler_params=_CP,
    )
    def phase2(
        rec0_hbm, rec1_hbm, out0_hbm, out1_hbm,
        slab, chunk, cl, cj,
    ):
        co = lax.axis_index("c")
        sid = lax.axis_index("s")
        jbase = sid * SLAB
        lanes = lax.iota(jnp.int32, 16)
        zeros16 = jnp.zeros((16,), jnp.float32)
        cl[pl.ds(0, 16)] = jnp.zeros((16,), jnp.int32)
        cj[pl.ds(0, 16)] = jnp.zeros((16,), jnp.int32)

        def zb(k, _):
            slab[k // (SLAB // 16), pl.ds((k % (SLAB // 16)) * 16, 16)] = zeros16
            return 0

        lax.fori_loop(0, (D * SLAB) // 16, zb, 0)

        def chunk_body(k, _):
            coff = pl.multiple_of(k * 128, 128)

            @pl.when(co == 0)
            def _():
                pltpu.sync_copy(rec0_hbm.at[:, pl.ds(coff, 128)], chunk)

            @pl.when(co == 1)
            def _():
                pltpu.sync_copy(rec1_hbm.at[:, pl.ds(coff, 128)], chunk)

            def grp(g, p):
                jv = plsc.bitcast(chunk[D, pl.ds(g * 16, 16)], jnp.int32)
                m = (jv >= jbase) & (jv < jbase + SLAB)
                cnt = jnp.sum(jnp.where(m, 1, 0))
                plsc.store_compressed(
                    cl.at[pl.ds(p, 16)], lanes + g * 16, mask=m
                )
                plsc.store_compressed(cj.at[pl.ds(p, 16)], jv - jbase, mask=m)
                return p + cnt

            p = lax.fori_loop(0, 8, grp, jnp.int32(0))

            def scat(g, _):
                lvec = cl[pl.ds(g * 16, 16)]
                jvec = cj[pl.ds(g * 16, 16)]
                wm = (lanes + g * 16) < p
                jvc = jnp.clip(jvec, 0, SLAB - 1)

                def frow(f, _):
                    fv = jnp.full((16,), 0, jnp.int32) + f
                    col = plsc.load_gather(chunk, [fv, lvec], mask=wm)
                    plsc.store_scatter(slab, [fv, jvc], col, mask=wm)
                    return 0

                lax.fori_loop(0, D, frow, 0)
                return 0

            lax.fori_loop(0, (p + 15) // 16, scat, 0)
            return 0

        lax.fori_loop(0, CAP // 128, chunk_body, 0)

        soff = pl.multiple_of(jbase, 128)

        @pl.when(co == 0)
        def _():
            pltpu.sync_copy(slab, out0_hbm.at[:, pl.ds(soff, SLAB)])

        @pl.when(co == 1)
        def _():
            pltpu.sync_copy(slab, out1_hbm.at[:, pl.ds(soff, SLAB)])

    return phase2


def kernel(idx, emb_weight):
    B = idx.shape[0]
    V, D = emb_weight.shape
    T = emb_weight.T

    info = plsc.get_sparse_core_info()
    NC, NS = info.num_cores, info.num_subcores
    NW = NC * NS
    CAP = B + NW * 128

    idx32 = idx.astype(jnp.int32)
    rec0, rec1 = _phase1(B, D, NC, NS, CAP, V)(idx32, T)
    o0, o1 = _phase2(B, D, NC, NS, CAP)(rec0, rec1)
    main = (o0 + o1).T

    vmain = (V // 128) * 128
    if vmain < V:
        tail_tab = lax.slice(emb_weight, (vmain, 0), (V, D))
        tidx = jnp.clip(idx32 - vmain, 0, V - vmain - 1)
        tail = jnp.take(tail_tab, tidx, axis=0)
        main = jnp.where((idx32 >= vmain)[:, None], tail, main)
    return main


# speedup vs baseline: 1.2916x; 1.2916x over previous
import functools

import jax
import jax.numpy as jnp
from jax import lax
from jax.experimental import pallas as pl
from jax.experimental.pallas import tpu as pltpu
from jax.experimental.pallas import tpu_sc as plsc

_MESH = plsc.VectorSubcoreMesh(core_axis_name="c", subcore_axis_name="s")
_CP = pltpu.CompilerParams(needs_layout_passes=False)


def _phase1(B, D, NC, NS, CAP, V):
    NW = NC * NS
    NCOLS = V // 128
    base_cols = NCOLS // NW
    extra = NCOLS - base_cols * NW
    NBLK = CAP // 128
    blk_base = NBLK // NS
    blk_extra = NBLK - blk_base * NS

    @functools.partial(
        pl.kernel,
        mesh=_MESH,
        out_type=(
            jax.ShapeDtypeStruct((D + 1, CAP), jnp.float32),
            jax.ShapeDtypeStruct((D + 1, CAP), jnp.float32),
        ),
        scratch_types=[
            pltpu.VMEM((2048,), jnp.int32),
            pltpu.VMEM((B,), jnp.int32),
            pltpu.VMEM((B,), jnp.int32),
            pltpu.VMEM((16,), jnp.int32),
            pltpu.VMEM((16,), jnp.int32),
            pltpu.VMEM((D, 128), jnp.float32),
            pltpu.VMEM((D, 128), jnp.float32),
            pltpu.VMEM((D + 1, 128), jnp.float32),
            pltpu.SMEM((1,), jnp.int32),
            pltpu.SemaphoreType.DMA,
            pltpu.SemaphoreType.DMA,
        ],
        compiler_params=_CP,
    )
    def phase1(
        idx_hbm, t_hbm, rec0_hbm, rec1_hbm,
        idx_v, lv, lj, t16l, t16j, blk0, blk1, batch,
        cnt_s, semb0, semb1,
    ):
        co = lax.axis_index("c")
        sid = lax.axis_index("s")
        wid = co * NS + sid
        c0 = wid * base_cols + jnp.minimum(wid, extra)
        ncol = base_cols + jnp.where(wid < extra, 1, 0)
        lanes = lax.iota(jnp.int32, 16)
        sentv = plsc.bitcast(jnp.full((16,), B, jnp.int32), jnp.float32)

        cnt_s[0] = 0
        t16l[pl.ds(0, 16)] = jnp.zeros((16,), jnp.int32)
        t16j[pl.ds(0, 16)] = jnp.zeros((16,), jnp.int32)

        def scan_win(wi, n):
            pltpu.sync_copy(idx_hbm.at[pl.ds(wi * 2048, 2048)], idx_v)

            def scan_body(k2, n):
                v = idx_v[pl.ds(k2 * 16, 16)]
                c = v // 128
                m = (c >= c0) & (c < c0 + ncol)
                cnt = jnp.sum(jnp.where(m, 1, 0))
                plsc.store_compressed(lv.at[pl.ds(n, 16)], v, mask=m)
                plsc.store_compressed(
                    lj.at[pl.ds(n, 16)], lanes + wi * 2048 + k2 * 16, mask=m
                )
                return n + cnt

            return lax.fori_loop(0, 2048 // 16, scan_body, n)

        nmatch = lax.fori_loop(0, B // 2048, scan_win, jnp.int32(0))

        def fill_sent(k, _):
            batch[D, pl.ds(k * 16, 16)] = sentv
            return 0

        lax.fori_loop(0, 128 // 16, fill_sent, 0)

        s0 = sid * blk_base + jnp.minimum(sid, blk_extra)
        nsb = blk_base + jnp.where(sid < blk_extra, 1, 0)

        def sent_blk(b, _):
            off = pl.multiple_of((s0 + b) * 128, 128)

            @pl.when(co == 0)
            def _():
                pltpu.sync_copy(
                    batch.at[:, pl.ds(0, 128)], rec0_hbm.at[:, pl.ds(off, 128)]
                )

            @pl.when(co == 1)
            def _():
                pltpu.sync_copy(
                    batch.at[:, pl.ds(0, 128)], rec1_hbm.at[:, pl.ds(off, 128)]
                )

            return 0

        lax.fori_loop(0, nsb, sent_blk, 0)
        plsc.subcore_barrier()

        nround = ((nmatch + 127) // 128) * 128
        my_base = plsc.fetch_and_add(cnt_s.at[0], nround, subcore_id=0)

        def flush(nflushed):
            off = pl.multiple_of(my_base + nflushed, 128)

            @pl.when(co == 0)
            def _():
                pltpu.sync_copy(
                    batch.at[:, pl.ds(0, 128)], rec0_hbm.at[:, pl.ds(off, 128)]
                )

            @pl.when(co == 1)
            def _():
                pltpu.sync_copy(
                    batch.at[:, pl.ds(0, 128)], rec1_hbm.at[:, pl.ds(off, 128)]
                )

        def fetch(t, buf, sem):
            off = pl.multiple_of((c0 + t) * 128, 128)
            pltpu.async_copy(t_hbm.at[:, pl.ds(off, 128)], buf, sem)

        fetch(0, blk0, semb0)

        def block_body(t, carry):
            def is_even():
                pltpu.make_async_copy(
                    t_hbm.at[:, pl.ds(pl.multiple_of(0, 128), 128)],
                    blk0, semb0,
                ).wait()

                @pl.when(t + 1 < ncol)
                def _():
                    fetch(t + 1, blk1, semb1)

                return 0

            def is_odd():
                pltpu.make_async_copy(
                    t_hbm.at[:, pl.ds(pl.multiple_of(0, 128), 128)],
                    blk1, semb1,
                ).wait()

                @pl.when(t + 1 < ncol)
                def _():
                    fetch(t + 1, blk0, semb0)

                return 0

            lax.cond(t % 2 == 0, is_even, is_odd)

            def collect(q, carry2):
                s, nflushed = carry2
                v = lv[pl.ds(q * 16, 16)]
                j = lj[pl.ds(q * 16, 16)]
                active = (lanes + q * 16) < nmatch
                m = active & (v // 128 == c0 + t)
                cnt = jnp.sum(jnp.where(m, 1, 0))
                plsc.store_compressed(t16l.at[pl.ds(0, 16)], v % 128, mask=m)
                plsc.store_compressed(t16j.at[pl.ds(0, 16)], j, mask=m)

                def write_part(shift):
                    lvec = t16l[pl.ds(0, 16)]
                    jvec = t16j[pl.ds(0, 16)]
                    dst = s + lanes - shift
                    wm = (lanes < cnt) & (dst >= 0) & (dst < 128)
                    dstc = jnp.clip(dst, 0, 127)

                    even = t % 2 == 0

                    def frow8(f8, _):
                        for df in range(8):
                            f = f8 * 8 + df
                            fv = jnp.full((16,), 0, jnp.int32) + f
                            c0l = plsc.load_gather(blk0, [fv, lvec], mask=wm & even)
                            c1l = plsc.load_gather(blk1, [fv, lvec], mask=wm & (~even))
                            col = jnp.where(even, c0l, c1l)
                            plsc.store_scatter(batch, [fv, dstc], col, mask=wm)
                        return 0

                    lax.fori_loop(0, D // 8, frow8, 0)
                    plsc.store_scatter(
                        batch, [jnp.full((16,), D, jnp.int32), dstc],
                        plsc.bitcast(jvec, jnp.float32), mask=wm,
                    )

                @pl.when(cnt > 0)
                def _():
                    write_part(0)

                def do_flush(carry3):
                    s, nflushed = carry3
                    flush(nflushed)

                    def resent(k, _):
                        batch[D, pl.ds(k * 16, 16)] = sentv
                        return 0

                    lax.fori_loop(0, 128 // 16, resent, 0)

                    @pl.when(s + cnt > 128)
                    def _():
                        write_part(128)

                    return s + cnt - 128, nflushed + 128

                return lax.cond(
                    s + cnt >= 128, do_flush,
                    lambda c: (c[0] + cnt, c[1]), (s, nflushed)
                )

            nq = (nmatch + 15) // 16
            return lax.fori_loop(0, nq, collect, carry)

        sfin, nfl = lax.fori_loop(
            0, ncol, block_body, (jnp.int32(0), jnp.int32(0))
        )

        @pl.when(sfin > 0)
        def _():
            flush(nfl)

    return phase1


def _phase2(B, D, NC, NS, CAP):
    SLAB = B // NS

    @functools.partial(
        pl.kernel,
        mesh=_MESH,
        out_type=(
            jax.ShapeDtypeStruct((D, B), jnp.float32),
            jax.ShapeDtypeStruct((D, B), jnp.float32),
        ),
        scratch_types=[
            pltpu.VMEM((D, SLAB), jnp.float32),
            pltpu.VMEM((D + 1, 128), jnp.float32),
            pltpu.VMEM((D + 1, 128), jnp.float32),
            pltpu.SemaphoreType.DMA,
            pltpu.SemaphoreType.DMA,
            pltpu.VMEM((144,), jnp.int32),
            pltpu.VMEM((144,), jnp.int32),
        ],
        compiler_params=_CP,
    )
    def phase2(
        rec0_hbm, rec1_hbm, out0_hbm, out1_hbm,
        slab, chunk0, chunk1, semc0, semc1, cl, cj,
    ):
        co = lax.axis_index("c")
        sid = lax.axis_index("s")
        jbase = sid * SLAB
        lanes = lax.iota(jnp.int32, 16)
        zeros16 = jnp.zeros((16,), jnp.float32)
        cl[pl.ds(0, 16)] = jnp.zeros((16,), jnp.int32)
        cj[pl.ds(0, 16)] = jnp.zeros((16,), jnp.int32)

        def zb(k, _):
            slab[k // (SLAB // 16), pl.ds((k % (SLAB // 16)) * 16, 16)] = zeros16
            return 0

        lax.fori_loop(0, (D * SLAB) // 16, zb, 0)

        def cfetch(k, buf, sem):
            coff = pl.multiple_of(k * 128, 128)

            @pl.when(co == 0)
            def _():
                pltpu.async_copy(rec0_hbm.at[:, pl.ds(coff, 128)], buf, sem)

            @pl.when(co == 1)
            def _():
                pltpu.async_copy(rec1_hbm.at[:, pl.ds(coff, 128)], buf, sem)

        cfetch(0, chunk0, semc0)
        NCHUNK = CAP // 128

        def chunk_body(k, _):
            even = k % 2 == 0

            def is_even():
                pltpu.make_async_copy(
                    rec0_hbm.at[:, pl.ds(pl.multiple_of(0, 128), 128)],
                    chunk0, semc0,
                ).wait()

                @pl.when(k + 1 < NCHUNK)
                def _():
                    cfetch(k + 1, chunk1, semc1)

                return 0

            def is_odd():
                pltpu.make_async_copy(
                    rec0_hbm.at[:, pl.ds(pl.multiple_of(0, 128), 128)],
                    chunk1, semc1,
                ).wait()

                @pl.when(k + 1 < NCHUNK)
                def _():
                    cfetch(k + 1, chunk0, semc0)

                return 0

            lax.cond(even, is_even, is_odd)

            def grp(g, p):
                jv0 = plsc.bitcast(chunk0[D, pl.ds(g * 16, 16)], jnp.int32)
                jv1 = plsc.bitcast(chunk1[D, pl.ds(g * 16, 16)], jnp.int32)
                jv = jnp.where(even, jv0, jv1)
                m = (jv >= jbase) & (jv < jbase + SLAB)
                cnt = jnp.sum(jnp.where(m, 1, 0))
                plsc.store_compressed(
                    cl.at[pl.ds(p, 16)], lanes + g * 16, mask=m
                )
                plsc.store_compressed(cj.at[pl.ds(p, 16)], jv - jbase, mask=m)
                return p + cnt

            p = lax.fori_loop(0, 8, grp, jnp.int32(0))

            def scat(g, _):
                lvec = cl[pl.ds(g * 16, 16)]
                jvec = cj[pl.ds(g * 16, 16)]
                wm = (lanes + g * 16) < p
                jvc = jnp.clip(jvec, 0, SLAB - 1)

                def frow8(f8, _):
                    for df in range(8):
                        f = f8 * 8 + df
                        fv = jnp.full((16,), 0, jnp.int32) + f
                        ce = plsc.load_gather(chunk0, [fv, lvec], mask=wm & even)
                        co_ = plsc.load_gather(chunk1, [fv, lvec], mask=wm & (~even))
                        col = jnp.where(even, ce, co_)
                        plsc.store_scatter(slab, [fv, jvc], col, mask=wm)
                    return 0

                lax.fori_loop(0, D // 8, frow8, 0)
                return 0

            lax.fori_loop(0, (p + 15) // 16, scat, 0)
            return 0

        lax.fori_loop(0, NCHUNK, chunk_body, 0)

        soff = pl.multiple_of(jbase, 128)

        @pl.when(co == 0)
        def _():
            pltpu.sync_copy(slab, out0_hbm.at[:, pl.ds(soff, SLAB)])

        @pl.when(co == 1)
        def _():
            pltpu.sync_copy(slab, out1_hbm.at[:, pl.ds(soff, SLAB)])

    return phase2


def kernel(idx, emb_weight):
    B = idx.shape[0]
    V, D = emb_weight.shape
    T = emb_weight.T

    info = plsc.get_sparse_core_info()
    NC, NS = info.num_cores, info.num_subcores
    NW = NC * NS
    CAP = B + NW * 128

    idx32 = idx.astype(jnp.int32)
    rec0, rec1 = _phase1(B, D, NC, NS, CAP, V)(idx32, T)
    o0, o1 = _phase2(B, D, NC, NS, CAP)(rec0, rec1)
    main = (o0 + o1).T

    vmain = (V // 128) * 128
    if vmain < V:
        tail_tab = lax.slice(emb_weight, (vmain, 0), (V, D))
        tidx = jnp.clip(idx32 - vmain, 0, V - vmain - 1)
        tail = jnp.take(tail_tab, tidx, axis=0)
        main = jnp.where((idx32 >= vmain)[:, None], tail, main)
    return main


# speedup vs baseline: 1.4757x; 1.1425x over previous
import functools

import jax
import jax.numpy as jnp
from jax import lax
from jax.experimental import pallas as pl
from jax.experimental.pallas import tpu as pltpu
from jax.experimental.pallas import tpu_sc as plsc


def kernel(idx, emb_weight):
    B = idx.shape[0]
    V, D = emb_weight.shape

    info = plsc.get_sparse_core_info()
    NC, NS = info.num_cores, info.num_subcores
    NW = NC * NS
    assert B % NW == 0
    b_per_w = B // NW

    mesh = plsc.VectorSubcoreMesh(core_axis_name="c", subcore_axis_name="s")

    @functools.partial(
        pl.kernel,
        mesh=mesh,
        out_type=jax.ShapeDtypeStruct((B, D), jnp.float32),
        scratch_types=[
            pltpu.VMEM((b_per_w,), jnp.int32),
            pltpu.VMEM((b_per_w, D), jnp.float32),
            pltpu.SemaphoreType.DMA,
        ],
        compiler_params=pltpu.CompilerParams(use_tc_tiling_on_sc=False),
    )
    def gather_kernel(idx_hbm, table_hbm, out_hbm, idx_v, rows_v, sem):
        wid = lax.axis_index("s") * NC + lax.axis_index("c")
        base = wid * b_per_w
        pltpu.sync_copy(idx_hbm.at[pl.ds(base, b_per_w)], idx_v)
        pltpu.async_copy(table_hbm.at[idx_v], rows_v, sem).wait()
        pltpu.sync_copy(rows_v, out_hbm.at[pl.ds(base, b_per_w)])

    return gather_kernel(idx.astype(jnp.int32), emb_weight)
